# baseline (speedup 1.0000x reference)
import functools

import jax
import jax.numpy as jnp
from jax import lax
from jax.experimental import pallas as pl
from jax.experimental.pallas import tpu as pltpu
from jax.experimental.pallas import tpu_sc as plsc

ROWS_PER_CHUNK = 128
_TR_UNROLL = 8


def _sc_untile_tables(t1, t2, tail1, tail2):
    N = t1.shape[1]
    N_AL = (N // 128) * 128
    N_TAIL = N - N_AL
    CHUNK = 12800
    n_full, rem = divmod(N_AL, CHUNK)
    widths = [CHUNK] * n_full + ([rem] if rem else [])
    info = plsc.get_sparse_core_info()
    NC, NS = info.num_cores, info.num_subcores

    mesh = plsc.VectorSubcoreMesh(core_axis_name="c", subcore_axis_name="s")
    out_type = [jax.ShapeDtypeStruct((16 * N,), jnp.float32)] * 2
    scratch_types = [
        pltpu.VMEM((CHUNK,), jnp.float32),
        pltpu.VMEM((CHUNK,), jnp.float32),
        pltpu.SemaphoreType.DMA,
        pltpu.SemaphoreType.DMA,
    ]

    @functools.partial(
        pl.kernel, mesh=mesh, out_type=out_type, scratch_types=scratch_types,
    )
    def k(in1, in2, tl1, tl2, o1, o2, buf_a, buf_b, sem_a, sem_b):
        wid = lax.axis_index("s") * NC + lax.axis_index("c")
        f = wid % 16
        bufs = (buf_a, buf_b)
        sems = (sem_a, sem_b)

        def work(t_ref, tl_ref, o_ref):
            c0 = 0
            pending = [None, None]
            for i, width in enumerate(widths):
                b = i % 2
                if pending[b] is not None:
                    pending[b].wait()
                pltpu.async_copy(t_ref.at[f, pl.ds(c0, width)],
                                 bufs[b].at[pl.ds(0, width)], sems[b]).wait()
                pending[b] = pltpu.async_copy(
                    bufs[b].at[pl.ds(0, width)],
                    o_ref.at[pl.ds(f * N + c0, width)], sems[b])
                c0 += width
            for p in pending:
                if p is not None:
                    p.wait()
            pltpu.sync_copy(tl_ref.at[pl.ds(f * N_TAIL, N_TAIL)],
                            buf_a.at[pl.ds(0, N_TAIL)])
            pltpu.sync_copy(buf_a.at[pl.ds(0, N_TAIL)],
                            o_ref.at[pl.ds(f * N + N_AL, N_TAIL)])

        @pl.when(wid < 16)
        def _():
            work(in1, tl1, o1)

        @pl.when(wid >= 16)
        def _():
            work(in2, tl2, o2)

    return k(t1, t2, tail1, tail2)


def _sc_gather_combined(big_fm, small_tables, idx_arrays, B):
    info = plsc.get_sparse_core_info()
    NC, NS = info.num_cores, info.num_subcores
    NW = NC * NS
    b_per_w = B // NW
    n_chunks = b_per_w // ROWS_PER_CHUNK
    n_big = len(big_fm)
    n_small = len(small_tables)
    n_tab = n_big + n_small
    GP = b_per_w + 9

    mesh = plsc.VectorSubcoreMesh(core_axis_name="c", subcore_axis_name="s")

    idx2 = [a.astype(jnp.int32).reshape(B // ROWS_PER_CHUNK, ROWS_PER_CHUNK)
            for a in idx_arrays]

    out_type = jax.ShapeDtypeStruct((B, 128), jnp.float32)
    scratch_types = (
        [pltpu.VMEM((n_chunks, ROWS_PER_CHUNK), jnp.int32) for _ in range(n_tab)]
        + [pltpu.VMEM((16, GP), jnp.float32) for _ in range(n_big)]
        + [pltpu.VMEM((b_per_w, 16), jnp.float32) for _ in range(n_tab)]
        + [pltpu.SemaphoreType.DMA]
    )

    @functools.partial(
        pl.kernel, mesh=mesh, out_type=out_type, scratch_types=scratch_types,
        compiler_params=pltpu.CompilerParams(
            use_tc_tiling_on_sc=False, needs_layout_passes=False),
    )
    def k(*refs):
        tabs = refs[:n_tab]
        idxs = refs[n_tab:2 * n_tab]
        out = refs[2 * n_tab]
        a = 2 * n_tab + 1
        idx_v = refs[a:a + n_tab]
        gf = refs[a + n_tab:a + n_tab + n_big]
        rows_v = refs[a + n_tab + n_big:a + 2 * n_tab + n_big]
        sem = refs[a + 2 * n_tab + n_big]

        wid = lax.axis_index("s") * NC + lax.axis_index("c")
        base = wid * b_per_w
        r0 = wid * n_chunks
        rows16 = lax.iota(jnp.int32, 16)
        rows_gp = rows16 * GP

        for t in range(n_tab):
            pltpu.sync_copy(idxs[t].at[pl.ds(r0, n_chunks)], idx_v[t])
        copies = []
        for t in range(n_big):
            for f in range(16):
                for j in range(n_chunks):
                    copies.append(pltpu.async_copy(
                        tabs[t].at[f].at[idx_v[t].at[j]],
                        gf[t].at[f, pl.ds(j * ROWS_PER_CHUNK,
                                          ROWS_PER_CHUNK)],
                        sem))
        for t in range(n_big, n_tab):
            for j in range(n_chunks):
                copies.append(pltpu.async_copy(
                    tabs[t].at[idx_v[t].at[j]],
                    rows_v[t].at[pl.ds(j * ROWS_PER_CHUNK, ROWS_PER_CHUNK)],
                    sem))
        for c in copies:
            c.wait()

        for t in range(n_big):
            gf_t = gf[t]
            rows_t = rows_v[t]

            @plsc.parallel_loop(0, b_per_w, unroll=_TR_UNROLL)
            def _(j, gf_t=gf_t, rows_t=rows_t):
                vals = plsc.load_gather(gf_t, [rows16, rows16 * 0 + j])
                rows_t[j] = vals

        for t in range(n_tab):
            pltpu.sync_copy(
                rows_v[t],
                out.at[pl.ds(base, b_per_w), pl.ds(t * 16, 16)])

    return k(*big_fm, *small_tables, *idx2)


def _mlp_body(g_ref, ctxt_ref,
              wc1t_ref, bc1_ref, wc2_ref, bc2_ref,
              w1g_ref, w1c_ref, b1_ref, w2_ref, b2_ref, w3_ref, b3_ref,
              out_ref):
    f32 = jnp.float32
    dn_rt = (((0,), (1,)), ((), ()))
    dn_ll = (((0,), (0,)), ((), ()))
    ctx_t = ctxt_ref[...]
    wc1t = wc1t_ref[...]
    h_t = jnp.maximum(
        wc1t[:, 0:1] * ctx_t[0:1, :] + wc1t[:, 1:2] * ctx_t[1:2, :]
        + bc1_ref[...], 0.0)
    cv_t = jnp.maximum(
        lax.dot_general(wc2_ref[...], h_t, dn_ll, preferred_element_type=f32)
        + bc2_ref[...], 0.0)
    gs = g_ref[...][:, 0:80]
    h1_t = jnp.maximum(
        lax.dot_general(w1g_ref[...], gs, dn_rt, preferred_element_type=f32)
        + lax.dot_general(w1c_ref[...], cv_t, dn_ll,
                          preferred_element_type=f32)
        + b1_ref[...], 0.0)
    h2_t = jnp.maximum(
        lax.dot_general(w2_ref[...], h1_t, dn_ll, preferred_element_type=f32)
        + b2_ref[...], 0.0)
    out_ref[...] = (
        lax.dot_general(w3_ref[...], h2_t, dn_ll, preferred_element_type=f32)
        + b3_ref[...])


def kernel(striker_ids, bowler_ids, bat_team_ids, bowl_team_ids, venue_ids,
           context, bat_emb, bowl_emb, team_emb, venue_emb,
           w_c1, b_c1, w_c2, b_c2, w1, b1, w2, b2, w3, b3):
    B = striker_ids.shape[0]
    V = bat_emb.shape[0]

    n_al = (V // 128) * 128
    bat_lin, bowl_lin = _sc_untile_tables(
        bat_emb.T, bowl_emb.T,
        bat_emb[n_al:].T.reshape(-1), bowl_emb[n_al:].T.reshape(-1))
    bat_fm = bat_lin.reshape(16, V)
    bowl_fm = bowl_lin.reshape(16, V)
    team_pad = jnp.pad(team_emb, ((0, 0), (0, 8)))
    venue_pad = jnp.pad(venue_emb, ((0, 0), (0, 8)))

    B2 = B // 2
    ids = [striker_ids, bowler_ids, bat_team_ids, bowl_team_ids, venue_ids]
    gs = [
        _sc_gather_combined(
            [bat_fm, bowl_fm],
            [team_pad, team_pad, venue_pad],
            [a[h * B2:(h + 1) * B2] for a in ids],
            B2)
        for h in range(2)
    ]

    pad8 = lambda m: jnp.pad(m, ((0, 8), (0, 0)))
    w1g = jnp.concatenate([
        w1[0:32],
        pad8(w1[32:40]), pad8(w1[40:48]), pad8(w1[48:56]),
    ], axis=0)
    w1c = w1[56:72]

    BK = 2048
    full = lambda s: pl.BlockSpec(s, lambda i: tuple(0 for _ in s))
    ctx_t = context.T

    outs = []
    for h in range(2):
        out_t = pl.pallas_call(
            _mlp_body,
            grid=(B2 // BK,),
            in_specs=[
                pl.BlockSpec((BK, 128), lambda i: (i, 0)),
                pl.BlockSpec((context.shape[1], BK), lambda i: (0, i)),
                full((32, 2)), full((32, 1)),
                full((32, 16)), full((16, 1)),
                full((80, 128)), full((16, 128)), full((128, 1)),
                full((128, 64)), full((64, 1)),
                full((64, 10)), full((10, 1)),
            ],
            out_specs=pl.BlockSpec((10, BK), lambda i: (0, i)),
            out_shape=jax.ShapeDtypeStruct((10, B2), jnp.float32),
        )(gs[h], ctx_t[:, h * B2:(h + 1) * B2],
          w_c1.T, b_c1.reshape(-1, 1), w_c2, b_c2.reshape(-1, 1),
          w1g, w1c, b1.reshape(-1, 1),
          w2, b2.reshape(-1, 1), w3, b3.reshape(-1, 1))
        outs.append(out_t)
    return jnp.concatenate(outs, axis=1).T


# speedup vs baseline: 1.1442x; 1.1442x over previous
import functools

import jax
import jax.numpy as jnp
from jax import lax
from jax.experimental import pallas as pl
from jax.experimental.pallas import tpu as pltpu
from jax.experimental.pallas import tpu_sc as plsc

ROWS_PER_CHUNK = 128
_TR_UNROLL = 8


def _sc_untile_table(t, tail):
    N = t.shape[1]
    N_AL = (N // 128) * 128
    N_TAIL = N - N_AL
    n_tiles = N_AL // 128
    HALF0 = ((n_tiles + 1) // 2) * 128
    HALF1 = N_AL - HALF0
    CHUNK = 12800

    def _widths(total):
        n_full, rem = divmod(total, CHUNK)
        return [CHUNK] * n_full + ([rem] if rem else [])

    info = plsc.get_sparse_core_info()
    NC, NS = info.num_cores, info.num_subcores

    mesh = plsc.VectorSubcoreMesh(core_axis_name="c", subcore_axis_name="s")
    out_type = jax.ShapeDtypeStruct((16 * N,), jnp.float32)
    scratch_types = [
        pltpu.VMEM((CHUNK,), jnp.float32),
        pltpu.VMEM((CHUNK,), jnp.float32),
        pltpu.SemaphoreType.DMA,
        pltpu.SemaphoreType.DMA,
    ]

    @functools.partial(
        pl.kernel, mesh=mesh, out_type=out_type, scratch_types=scratch_types,
    )
    def k(t_ref, tl_ref, o_ref, buf_a, buf_b, sem_a, sem_b):
        wid = lax.axis_index("s") * NC + lax.axis_index("c")
        f = wid % 16
        bufs = (buf_a, buf_b)
        sems = (sem_a, sem_b)

        def work(col0, widths, do_tail):
            c0 = col0
            pending = [None, None]
            for i, width in enumerate(widths):
                b = i % 2
                if pending[b] is not None:
                    pending[b].wait()
                pltpu.async_copy(t_ref.at[f, pl.ds(c0, width)],
                                 bufs[b].at[pl.ds(0, width)], sems[b]).wait()
                pending[b] = pltpu.async_copy(
                    bufs[b].at[pl.ds(0, width)],
                    o_ref.at[pl.ds(f * N + c0, width)], sems[b])
                c0 += width
            for p in pending:
                if p is not None:
                    p.wait()
            if do_tail:
                pltpu.sync_copy(tl_ref.at[pl.ds(f * N_TAIL, N_TAIL)],
                                buf_a.at[pl.ds(0, N_TAIL)])
                pltpu.sync_copy(buf_a.at[pl.ds(0, N_TAIL)],
                                o_ref.at[pl.ds(f * N + N_AL, N_TAIL)])

        @pl.when(wid < 16)
        def _():
            work(0, _widths(HALF0), True)

        @pl.when(wid >= 16)
        def _():
            work(HALF0, _widths(HALF1), False)

    return k(t, tail)


def _sc_gather_combined(big_fm, small_tables, idx_arrays, B):
    info = plsc.get_sparse_core_info()
    NC, NS = info.num_cores, info.num_subcores
    NW = NC * NS
    b_per_w = B // NW
    n_chunks = b_per_w // ROWS_PER_CHUNK
    n_big = len(big_fm)
    n_small = len(small_tables)
    n_tab = n_big + n_small
    GP = b_per_w + 9

    mesh = plsc.VectorSubcoreMesh(core_axis_name="c", subcore_axis_name="s")

    idx2 = [a.astype(jnp.int32).reshape(B // ROWS_PER_CHUNK, ROWS_PER_CHUNK)
            for a in idx_arrays]

    out_type = jax.ShapeDtypeStruct((B, 128), jnp.float32)
    scratch_types = (
        [pltpu.VMEM((n_chunks, ROWS_PER_CHUNK), jnp.int32) for _ in range(n_tab)]
        + [pltpu.VMEM((16, GP), jnp.float32) for _ in range(n_big)]
        + [pltpu.VMEM((b_per_w, 16), jnp.float32) for _ in range(n_tab)]
        + [pltpu.SemaphoreType.DMA]
    )

    @functools.partial(
        pl.kernel, mesh=mesh, out_type=out_type, scratch_types=scratch_types,
        compiler_params=pltpu.CompilerParams(
            use_tc_tiling_on_sc=False, needs_layout_passes=False),
    )
    def k(*refs):
        tabs = refs[:n_tab]
        idxs = refs[n_tab:2 * n_tab]
        out = refs[2 * n_tab]
        a = 2 * n_tab + 1
        idx_v = refs[a:a + n_tab]
        gf = refs[a + n_tab:a + n_tab + n_big]
        rows_v = refs[a + n_tab + n_big:a + 2 * n_tab + n_big]
        sem = refs[a + 2 * n_tab + n_big]

        wid = lax.axis_index("s") * NC + lax.axis_index("c")
        base = wid * b_per_w
        r0 = wid * n_chunks
        rows16 = lax.iota(jnp.int32, 16)
        rows_gp = rows16 * GP

        for t in range(n_tab):
            pltpu.sync_copy(idxs[t].at[pl.ds(r0, n_chunks)], idx_v[t])
        copies = []
        for t in range(n_big):
            for f in range(16):
                for j in range(n_chunks):
                    copies.append(pltpu.async_copy(
                        tabs[t].at[f].at[idx_v[t].at[j]],
                        gf[t].at[f, pl.ds(j * ROWS_PER_CHUNK,
                                          ROWS_PER_CHUNK)],
                        sem))
        for t in range(n_big, n_tab):
            for j in range(n_chunks):
                copies.append(pltpu.async_copy(
                    tabs[t].at[idx_v[t].at[j]],
                    rows_v[t].at[pl.ds(j * ROWS_PER_CHUNK, ROWS_PER_CHUNK)],
                    sem))
        for c in copies:
            c.wait()

        for t in range(n_big):
            gf_t = gf[t]
            rows_t = rows_v[t]

            @plsc.parallel_loop(0, b_per_w, unroll=_TR_UNROLL)
            def _(j, gf_t=gf_t, rows_t=rows_t):
                vals = plsc.load_gather(gf_t, [rows16, rows16 * 0 + j])
                rows_t[j] = vals

        for t in range(n_tab):
            pltpu.sync_copy(
                rows_v[t],
                out.at[pl.ds(base, b_per_w), pl.ds(t * 16, 16)])

    return k(*big_fm, *small_tables, *idx2)


def _mlp_body(g_ref, ctxt_ref,
              wc1t_ref, bc1_ref, wc2_ref, bc2_ref,
              w1g_ref, w1c_ref, b1_ref, w2_ref, b2_ref, w3_ref, b3_ref,
              out_ref):
    f32 = jnp.float32
    dn_rt = (((0,), (1,)), ((), ()))
    dn_ll = (((0,), (0,)), ((), ()))
    ctx_t = ctxt_ref[...]
    wc1t = wc1t_ref[...]
    h_t = jnp.maximum(
        wc1t[:, 0:1] * ctx_t[0:1, :] + wc1t[:, 1:2] * ctx_t[1:2, :]
        + bc1_ref[...], 0.0)
    cv_t = jnp.maximum(
        lax.dot_general(wc2_ref[...], h_t, dn_ll, preferred_element_type=f32)
        + bc2_ref[...], 0.0)
    gs = g_ref[...][:, 0:80]
    h1_t = jnp.maximum(
        lax.dot_general(w1g_ref[...], gs, dn_rt, preferred_element_type=f32)
        + lax.dot_general(w1c_ref[...], cv_t, dn_ll,
                          preferred_element_type=f32)
        + b1_ref[...], 0.0)
    h2_t = jnp.maximum(
        lax.dot_general(w2_ref[...], h1_t, dn_ll, preferred_element_type=f32)
        + b2_ref[...], 0.0)
    out_ref[...] = (
        lax.dot_general(w3_ref[...], h2_t, dn_ll, preferred_element_type=f32)
        + b3_ref[...])


def kernel(striker_ids, bowler_ids, bat_team_ids, bowl_team_ids, venue_ids,
           context, bat_emb, bowl_emb, team_emb, venue_emb,
           w_c1, b_c1, w_c2, b_c2, w1, b1, w2, b2, w3, b3):
    B = striker_ids.shape[0]
    V = bat_emb.shape[0]

    n_al = (V // 128) * 128
    bat_fm = bat_emb.T.reshape(-1).reshape(16, V)
    bowl_lin = _sc_untile_table(bowl_emb.T, bowl_emb[n_al:].T.reshape(-1))
    bowl_fm = bowl_lin.reshape(16, V)
    team_pad = jnp.pad(team_emb, ((0, 0), (0, 8)))
    venue_pad = jnp.pad(venue_emb, ((0, 0), (0, 8)))

    g = _sc_gather_combined(
        [bat_fm, bowl_fm],
        [team_pad, team_pad, venue_pad],
        [striker_ids, bowler_ids, bat_team_ids, bowl_team_ids, venue_ids],
        B)

    pad8 = lambda m: jnp.pad(m, ((0, 8), (0, 0)))
    w1g = jnp.concatenate([
        w1[0:32],
        pad8(w1[32:40]), pad8(w1[40:48]), pad8(w1[48:56]),
    ], axis=0)
    w1c = w1[56:72]

    BK = 2048
    full = lambda s: pl.BlockSpec(s, lambda i: tuple(0 for _ in s))

    out_t = pl.pallas_call(
        _mlp_body,
        grid=(B // BK,),
        in_specs=[
            pl.BlockSpec((BK, 128), lambda i: (i, 0)),
            pl.BlockSpec((context.shape[1], BK), lambda i: (0, i)),
            full((32, 2)), full((32, 1)),
            full((32, 16)), full((16, 1)),
            full((80, 128)), full((16, 128)), full((128, 1)),
            full((128, 64)), full((64, 1)),
            full((64, 10)), full((10, 1)),
        ],
        out_specs=pl.BlockSpec((10, BK), lambda i: (0, i)),
        out_shape=jax.ShapeDtypeStruct((10, B), jnp.float32),
    )(g, context.T,
      w_c1.T, b_c1.reshape(-1, 1), w_c2, b_c2.reshape(-1, 1),
      w1g, w1c, b1.reshape(-1, 1),
      w2, b2.reshape(-1, 1), w3, b3.reshape(-1, 1))
    return out_t.T
